# baseline (speedup 1.0000x reference)
import jax
import jax.numpy as jnp
from jax.experimental import pallas as pl


def _elu_body(x_ref, o_ref):
    v = x_ref[...]
    o_ref[...] = jnp.where(v > 0, v, jnp.exp(v) - 1.0)


def _elu(x):
    n, c = x.shape
    blk = 1000
    return pl.pallas_call(
        _elu_body,
        grid=(n // blk,),
        in_specs=[pl.BlockSpec((blk, c), lambda i: (i, 0))],
        out_specs=pl.BlockSpec((blk, c), lambda i: (i, 0)),
        out_shape=jax.ShapeDtypeStruct(x.shape, x.dtype),
    )(x)


def _gat(x, edge_index, W, a_s, a_d, b, heads, outd, concat):
    N = x.shape[0]
    loop = jnp.arange(N, dtype=edge_index.dtype)
    src = jnp.concatenate([edge_index[0], loop])
    dst = jnp.concatenate([edge_index[1], loop])
    h = (x @ W).reshape(N, heads, outd)
    e = jnp.sum(h * a_s[None, :, :], axis=-1)[src] + jnp.sum(h * a_d[None, :, :], axis=-1)[dst]
    e = jnp.where(e > 0, e, 0.2 * e)
    m = jax.ops.segment_max(e, dst, num_segments=N)
    ex = jnp.exp(e - m[dst])
    s = jax.ops.segment_sum(ex, dst, num_segments=N)
    alpha = ex / (s[dst] + 1e-16)
    out = jax.ops.segment_sum(h[src] * alpha[:, :, None], dst, num_segments=N)
    out = out.reshape(N, heads * outd) if concat else jnp.mean(out, axis=1)
    return out + b


def kernel(x, edge_index, batch, params):
    p = params
    x = _elu(_gat(x, edge_index, p['W1'], p['as1'], p['ad1'], p['b1'], 8, 128, True))
    x = _elu(_gat(x, edge_index, p['W2'], p['as2'], p['ad2'], p['b2'], 4, 128, True))
    x = _elu(_gat(x, edge_index, p['W3'], p['as3'], p['ad3'], p['b3'], 1, 128, False))
    sm = jax.ops.segment_sum(x, batch, num_segments=64)
    cnt = jax.ops.segment_sum(jnp.ones((x.shape[0],), x.dtype), batch, num_segments=64)
    g = sm / jnp.maximum(cnt, 1.0)[:, None]
    outs = []
    for i in range(8):
        hp = p['slots'][str(i)]
        h = jnp.maximum(g @ hp['w1'] + hp['b1'], 0.0)
        if 'w3' in hp:
            h = jnp.maximum(h @ hp['w2'] + hp['b2'], 0.0)
            h = h @ hp['w3'] + hp['b3']
        else:
            h = h @ hp['w2'] + hp['b2']
        outs.append(h)
    return tuple(outs)


# speedup vs baseline: 27.2401x; 27.2401x over previous
import functools

import jax
import jax.numpy as jnp
from jax import lax
from jax.experimental import pallas as pl
from jax.experimental.pallas import tpu as pltpu
from jax.experimental.pallas import tpu_sc as plsc

N = 10000
NP = 10240
C = 96
NT = 16
EPAD = 331776
NPT = NP // NT


def _edge_body(cfg, h2, as_t, ad_t, shift, ech_h, zh, zs,
               out_h, out_s,
               as_v, ad_v, ed_v, gidx_v, dstc_v, w_v, rows_v, shv,
               acc_sh, s_sh, gsem, esem):
    H, HC, NCH, e_tot, split32 = cfg
    c = lax.axis_index("c")
    s = lax.axis_index("s")
    if split32:
        tile = s * 2 + c
    else:
        tile = s
    ch0 = tile * NCH
    pltpu.sync_copy(shift, shv)
    shift_vec = shv[...]
    nsl = pl.ds(s * NPT, NPT)
    nr = C // 16

    for hi in range(HC):
        if split32:
            k = 0
            slot = c
        else:
            k = c * HC + hi
            slot = k
        kn = k * N
        pltpu.sync_copy(as_t.at[k], as_v)
        pltpu.sync_copy(ad_t.at[k], ad_v)
        pltpu.sync_copy(zh, acc_sh.at[nsl])
        pltpu.sync_copy(zs, s_sh.at[nsl])
        plsc.subcore_barrier()

        def prep(ch, b):
            for r in range(nr):
                sl = pl.ds(r * 16, 16)
                sv = ed_v[b, 0, sl]
                dv = ed_v[b, 1, sl]
                gidx_v[b, sl] = sv + kn
                dstc_v[b, sl] = dv
                e = plsc.load_gather(as_v, [sv]) + plsc.load_gather(ad_v, [dv])
                e = jnp.maximum(e, 0.2 * e)
                w = jnp.exp(e - shift_vec)
                gid = (ch0 + ch) * C + r * 16 + lax.iota(jnp.int32, 16)
                w_v[b, sl] = jnp.where(gid < e_tot, w, 0.0)
            pltpu.async_copy(h2.at[gidx_v.at[b]], rows_v.at[b], gsem.at[b])

        def process(ch, b):
            def scale(e_i, carry):
                wv = plsc.load_gather(
                    w_v.at[b], [jnp.full((16,), e_i, jnp.int32)])
                for r8 in range(8):
                    sl = pl.ds(r8 * 16, 16)
                    rows_v[b, e_i, sl] = rows_v[b, e_i, sl] * wv
                return carry

            lax.fori_loop(0, C, scale, 0)
            pltpu.sync_copy(rows_v.at[b], acc_sh.at[dstc_v.at[b]], add=True)
            pltpu.sync_copy(w_v.at[b], s_sh.at[dstc_v.at[b]], add=True)

        pltpu.sync_copy(ech_h.at[ch0], ed_v.at[0])
        prep(0, 0)
        pltpu.async_copy(ech_h.at[ch0 + 1], ed_v.at[1], esem.at[1])

        def chunk2(ch2):
            for b in range(2):
                ch = ch2 + b
                nb = 1 - b

                @pl.when(ch + 2 < NCH)
                def _():
                    pltpu.async_copy(ech_h.at[ch0 + ch + 2], ed_v.at[b],
                                     esem.at[b])

                @pl.when(ch + 1 < NCH)
                def _():
                    pltpu.make_async_copy(ech_h.at[ch0 + ch + 1],
                                          ed_v.at[nb], esem.at[nb]).wait()
                    prep(ch + 1, nb)

                pltpu.make_async_copy(h2.at[gidx_v.at[b]], rows_v.at[b],
                                      gsem.at[b]).wait()
                process(ch, b)

        pl.loop(0, NCH, step=2)(chunk2)
        plsc.subcore_barrier()
        pltpu.sync_copy(acc_sh.at[nsl], out_h.at[slot, nsl])
        pltpu.sync_copy(s_sh.at[nsl], out_s.at[slot, nsl])
        plsc.subcore_barrier()


@functools.cache
def _edge_kernel(H, e_tot, split32):
    HC = 1 if split32 else H // 2
    EPT = EPAD // 32 if split32 else EPAD // 16
    NCH = EPT // C
    S = 2 if split32 else H
    mesh = plsc.VectorSubcoreMesh(core_axis_name="c", subcore_axis_name="s")
    body = functools.partial(_edge_body, (H, HC, NCH, e_tot, split32))
    return pl.kernel(
        body,
        out_type=(jax.ShapeDtypeStruct((S, NP, 128), jnp.float32),
                  jax.ShapeDtypeStruct((S, NP), jnp.float32)),
        mesh=mesh,
        compiler_params=pltpu.CompilerParams(needs_layout_passes=False),
        scratch_types=[
            pltpu.VMEM((N,), jnp.float32),
            pltpu.VMEM((N,), jnp.float32),
            pltpu.VMEM((2, 2, C), jnp.int32),
            pltpu.VMEM((2, C), jnp.int32),
            pltpu.VMEM((2, C), jnp.int32),
            pltpu.VMEM((2, C), jnp.float32),
            pltpu.VMEM((2, C, 128), jnp.float32),
            pltpu.VMEM((16,), jnp.float32),
            pltpu.VMEM_SHARED((NP, 128), jnp.float32),
            pltpu.VMEM_SHARED((NP,), jnp.float32),
            pltpu.SemaphoreType.DMA((2,)),
            pltpu.SemaphoreType.DMA((2,)),
        ],
    )


def _elu_body(x_ref, o_ref):
    v = x_ref[...]
    o_ref[...] = jnp.where(v > 0, v, jnp.exp(v) - 1.0)


def _elu(x):
    n, ccol = x.shape
    blk = 1000
    return pl.pallas_call(
        _elu_body,
        grid=(n // blk,),
        in_specs=[pl.BlockSpec((blk, ccol), lambda i: (i, 0))],
        out_specs=pl.BlockSpec((blk, ccol), lambda i: (i, 0)),
        out_shape=jax.ShapeDtypeStruct(x.shape, x.dtype),
    )(x)


def kernel(x, edge_index, batch, params):
    p = params
    n_nodes = x.shape[0]
    e_in = edge_index.shape[1]
    e_tot = e_in + n_nodes
    loop = jnp.arange(n_nodes, dtype=jnp.int32)
    src = jnp.concatenate([edge_index[0].astype(jnp.int32), loop,
                           jnp.zeros((EPAD - e_tot,), jnp.int32)])
    dst = jnp.concatenate([edge_index[1].astype(jnp.int32), loop,
                           jnp.zeros((EPAD - e_tot,), jnp.int32)])
    ech = jnp.stack([src.reshape(EPAD // C, C),
                     dst.reshape(EPAD // C, C)], axis=1)
    zh = jnp.zeros((NPT, 128), jnp.float32)
    zs = jnp.zeros((NPT,), jnp.float32)

    def gat(xin, W, a_s, a_d, b, H):
        h = xin @ W
        hh = h.reshape(n_nodes, H, 128)
        as_n = jnp.sum(hh * a_s[None, :, :], axis=-1).T
        ad_n = jnp.sum(hh * a_d[None, :, :], axis=-1).T
        m = jnp.max(as_n) + jnp.max(ad_n)
        shift = jnp.full((16,), jnp.maximum(m, 0.2 * m), jnp.float32)
        h2 = hh.transpose(1, 0, 2).reshape(H * n_nodes, 128)
        split32 = (H == 1)
        out_h, out_s = _edge_kernel(H, e_tot, split32)(
            h2, as_n, ad_n, shift, ech, zh, zs)
        out_h = out_h[:, :n_nodes]
        out_s = out_s[:, :n_nodes]
        if split32:
            out = (out_h[0] + out_h[1]) / (out_s[0] + out_s[1])[:, None]
        else:
            out = out_h / out_s[:, :, None]
            out = out.transpose(1, 0, 2).reshape(n_nodes, H * 128)
        return out + b

    h1 = _elu(gat(x, p['W1'], p['as1'], p['ad1'], p['b1'], 8))
    h2 = _elu(gat(h1, p['W2'], p['as2'], p['ad2'], p['b2'], 4))
    h3 = _elu(gat(h2, p['W3'], p['as3'], p['ad3'], p['b3'], 1))
    sm = jax.ops.segment_sum(h3, batch, num_segments=64)
    cnt = jax.ops.segment_sum(jnp.ones((n_nodes,), h3.dtype), batch,
                              num_segments=64)
    g = sm / jnp.maximum(cnt, 1.0)[:, None]
    outs = []
    for i in range(8):
        hp = p['slots'][str(i)]
        hmlp = jnp.maximum(g @ hp['w1'] + hp['b1'], 0.0)
        if 'w3' in hp:
            hmlp = jnp.maximum(hmlp @ hp['w2'] + hp['b2'], 0.0)
            hmlp = hmlp @ hp['w3'] + hp['b3']
        else:
            hmlp = hmlp @ hp['w2'] + hp['b2']
        outs.append(hmlp)
    return tuple(outs)


# speedup vs baseline: 32.7820x; 1.2034x over previous
import functools

import jax
import jax.numpy as jnp
from jax import lax
from jax.experimental import pallas as pl
from jax.experimental.pallas import tpu as pltpu
from jax.experimental.pallas import tpu_sc as plsc

N = 10000
NP = 10240
C = 96
NT = 16
EPAD = 331776
NPT = NP // NT


def _edge_body(cfg, h2, as_t, ad_t, shift, ech_h, zh, zs,
               out_h, out_s,
               as_v, ad_v, ed_v, gidx_v, dstc_v, w_v, rows_v, shv,
               acc_sh, s_sh, gsem, esem):
    H, HC, NCH, e_tot, split32 = cfg
    c = lax.axis_index("c")
    s = lax.axis_index("s")
    if split32:
        tile = s * 2 + c
    else:
        tile = s
    ch0 = tile * NCH
    pltpu.sync_copy(shift, shv)
    shift_vec = shv[...]
    nsl = pl.ds(s * NPT, NPT)
    nr = C // 16

    for hi in range(HC):
        if split32:
            k = 0
            slot = c
        else:
            k = c * HC + hi
            slot = k
        kn = k * N
        pltpu.sync_copy(as_t.at[k], as_v)
        pltpu.sync_copy(ad_t.at[k], ad_v)
        pltpu.sync_copy(zh, acc_sh.at[nsl])
        pltpu.sync_copy(zs, s_sh.at[nsl])
        plsc.subcore_barrier()

        def prep(ch, b):
            for r in range(nr):
                sl = pl.ds(r * 16, 16)
                sv = ed_v[b, 0, sl]
                dv = ed_v[b, 1, sl]
                gidx_v[b, sl] = sv + kn
                dstc_v[b, sl] = dv
                e = plsc.load_gather(as_v, [sv]) + plsc.load_gather(ad_v, [dv])
                e = jnp.maximum(e, 0.2 * e)
                w = jnp.exp(e - shift_vec)
                gid = (ch0 + ch) * C + r * 16 + lax.iota(jnp.int32, 16)
                w_v[b, sl] = jnp.where(gid < e_tot, w, 0.0)
            pltpu.async_copy(h2.at[gidx_v.at[b]], rows_v.at[b], gsem.at[b])

        def process(ch, b):
            @plsc.parallel_loop(0, C, 1, unroll=4)
            def _(e_i):
                wv = plsc.load_gather(
                    w_v.at[b], [jnp.full((16,), e_i, jnp.int32)])
                for r8 in range(8):
                    sl = pl.ds(r8 * 16, 16)
                    rows_v[b, e_i, sl] = rows_v[b, e_i, sl] * wv
            pltpu.sync_copy(rows_v.at[b], acc_sh.at[dstc_v.at[b]], add=True)
            pltpu.sync_copy(w_v.at[b], s_sh.at[dstc_v.at[b]], add=True)

        pltpu.sync_copy(ech_h.at[ch0], ed_v.at[0])
        prep(0, 0)
        pltpu.async_copy(ech_h.at[ch0 + 1], ed_v.at[1], esem.at[1])

        def chunk2(ch2):
            for b in range(2):
                ch = ch2 + b
                nb = 1 - b

                @pl.when(ch + 2 < NCH)
                def _():
                    pltpu.async_copy(ech_h.at[ch0 + ch + 2], ed_v.at[b],
                                     esem.at[b])

                @pl.when(ch + 1 < NCH)
                def _():
                    pltpu.make_async_copy(ech_h.at[ch0 + ch + 1],
                                          ed_v.at[nb], esem.at[nb]).wait()
                    prep(ch + 1, nb)

                pltpu.make_async_copy(h2.at[gidx_v.at[b]], rows_v.at[b],
                                      gsem.at[b]).wait()
                process(ch, b)

        pl.loop(0, NCH, step=2)(chunk2)
        plsc.subcore_barrier()
        pltpu.sync_copy(acc_sh.at[nsl], out_h.at[slot, nsl])
        pltpu.sync_copy(s_sh.at[nsl], out_s.at[slot, nsl])
        plsc.subcore_barrier()


@functools.cache
def _edge_kernel(H, e_tot, split32):
    HC = 1 if split32 else H // 2
    EPT = EPAD // 32 if split32 else EPAD // 16
    NCH = EPT // C
    S = 2 if split32 else H
    mesh = plsc.VectorSubcoreMesh(core_axis_name="c", subcore_axis_name="s")
    body = functools.partial(_edge_body, (H, HC, NCH, e_tot, split32))
    return pl.kernel(
        body,
        out_type=(jax.ShapeDtypeStruct((S, NP, 128), jnp.float32),
                  jax.ShapeDtypeStruct((S, NP), jnp.float32)),
        mesh=mesh,
        compiler_params=pltpu.CompilerParams(needs_layout_passes=False),
        scratch_types=[
            pltpu.VMEM((N,), jnp.float32),
            pltpu.VMEM((N,), jnp.float32),
            pltpu.VMEM((2, 2, C), jnp.int32),
            pltpu.VMEM((2, C), jnp.int32),
            pltpu.VMEM((2, C), jnp.int32),
            pltpu.VMEM((2, C), jnp.float32),
            pltpu.VMEM((2, C, 128), jnp.float32),
            pltpu.VMEM((16,), jnp.float32),
            pltpu.VMEM_SHARED((NP, 128), jnp.float32),
            pltpu.VMEM_SHARED((NP,), jnp.float32),
            pltpu.SemaphoreType.DMA((2,)),
            pltpu.SemaphoreType.DMA((2,)),
        ],
    )


def _elu_body(x_ref, o_ref):
    v = x_ref[...]
    o_ref[...] = jnp.where(v > 0, v, jnp.exp(v) - 1.0)


def _elu(x):
    n, ccol = x.shape
    blk = 1000
    return pl.pallas_call(
        _elu_body,
        grid=(n // blk,),
        in_specs=[pl.BlockSpec((blk, ccol), lambda i: (i, 0))],
        out_specs=pl.BlockSpec((blk, ccol), lambda i: (i, 0)),
        out_shape=jax.ShapeDtypeStruct(x.shape, x.dtype),
    )(x)


def kernel(x, edge_index, batch, params):
    p = params
    n_nodes = x.shape[0]
    e_in = edge_index.shape[1]
    e_tot = e_in + n_nodes
    loop = jnp.arange(n_nodes, dtype=jnp.int32)
    src = jnp.concatenate([edge_index[0].astype(jnp.int32), loop,
                           jnp.zeros((EPAD - e_tot,), jnp.int32)])
    dst = jnp.concatenate([edge_index[1].astype(jnp.int32), loop,
                           jnp.zeros((EPAD - e_tot,), jnp.int32)])
    ech = jnp.stack([src.reshape(EPAD // C, C),
                     dst.reshape(EPAD // C, C)], axis=1)
    zh = jnp.zeros((NPT, 128), jnp.float32)
    zs = jnp.zeros((NPT,), jnp.float32)

    def gat(xin, W, a_s, a_d, b, H):
        h = xin @ W
        hh = h.reshape(n_nodes, H, 128)
        as_n = jnp.sum(hh * a_s[None, :, :], axis=-1).T
        ad_n = jnp.sum(hh * a_d[None, :, :], axis=-1).T
        m = jnp.max(as_n) + jnp.max(ad_n)
        shift = jnp.full((16,), jnp.maximum(m, 0.2 * m), jnp.float32)
        h2 = hh.transpose(1, 0, 2).reshape(H * n_nodes, 128)
        split32 = (H == 1)
        out_h, out_s = _edge_kernel(H, e_tot, split32)(
            h2, as_n, ad_n, shift, ech, zh, zs)
        out_h = out_h[:, :n_nodes]
        out_s = out_s[:, :n_nodes]
        if split32:
            out = (out_h[0] + out_h[1]) / (out_s[0] + out_s[1])[:, None]
        else:
            out = out_h / out_s[:, :, None]
            out = out.transpose(1, 0, 2).reshape(n_nodes, H * 128)
        return out + b

    h1 = _elu(gat(x, p['W1'], p['as1'], p['ad1'], p['b1'], 8))
    h2 = _elu(gat(h1, p['W2'], p['as2'], p['ad2'], p['b2'], 4))
    h3 = _elu(gat(h2, p['W3'], p['as3'], p['ad3'], p['b3'], 1))
    sm = jax.ops.segment_sum(h3, batch, num_segments=64)
    cnt = jax.ops.segment_sum(jnp.ones((n_nodes,), h3.dtype), batch,
                              num_segments=64)
    g = sm / jnp.maximum(cnt, 1.0)[:, None]
    outs = []
    for i in range(8):
        hp = p['slots'][str(i)]
        hmlp = jnp.maximum(g @ hp['w1'] + hp['b1'], 0.0)
        if 'w3' in hp:
            hmlp = jnp.maximum(hmlp @ hp['w2'] + hp['b2'], 0.0)
            hmlp = hmlp @ hp['w3'] + hp['b3']
        else:
            hmlp = hmlp @ hp['w2'] + hp['b2']
        outs.append(hmlp)
    return tuple(outs)
